# baseline (speedup 1.0000x reference)
import jax
import jax.numpy as jnp
from jax import lax
from jax.experimental import pallas as pl
from jax.experimental.pallas import tpu as pltpu
from jax.experimental.pallas import tpu_sc as plsc

N = 10000
IN_CH = 128
HID = 64
OUT_CH = 64
HEADS = 8
E_RAW = 320000
E_TOT = E_RAW + N
PAD_N = 10240
NC = 2
NS = 16
NW = NC * NS
B = 128
NB = 81
NBB = NB * B
E_PAD = NW * NBB
R16 = PAD_N // NS
NBUF = 4

_MESH = plsc.VectorSubcoreMesh(
    core_axis_name="c", subcore_axis_name="s", num_cores=NC, num_subcores=NS)
_SC_PARAMS = pltpu.CompilerParams(use_tc_tiling_on_sc=False, needs_layout_passes=False)


def _sc_attn_body(src_hbm, dst_hbm, asrc_hbm, adst_hbm, z16_hbm,
                  expa_hbm, denom_hbm,
                  src_v, dst_v, rs_v, rd_v, ex_v, acc, sem1, sem2):
    c = lax.axis_index("c")
    s = lax.axis_index("s")
    wid = c * NS + s
    r0 = s * R16
    base0 = wid * NBB
    pltpu.sync_copy(src_hbm.at[pl.ds(base0, NBB)], src_v)
    pltpu.sync_copy(dst_hbm.at[pl.ds(base0, NBB)], dst_v)

    def _gathers(b, p):
        return (
            pltpu.make_async_copy(
                asrc_hbm.at[src_v.at[pl.ds(b * B, B)]], rs_v.at[p], sem1.at[p]),
            pltpu.make_async_copy(
                adst_hbm.at[dst_v.at[pl.ds(b * B, B)]], rd_v.at[p], sem2.at[p]),
        )

    for i in range(NBUF):
        for cp in _gathers(i, i):
            cp.start()
    pltpu.sync_copy(z16_hbm.at[pl.ds(r0, R16)], acc.at[pl.ds(r0, R16)])
    plsc.subcore_barrier()

    def batch(b, carry):
        p = lax.rem(b, NBUF)
        for cp in _gathers(b, p):
            cp.wait()

        def edge(e, carry2):
            a = rs_v[p, e] + rd_v[p, e]
            a = jnp.maximum(a, 0.2 * a)
            ex_v[e] = jnp.exp(a)
            return carry2
        lax.fori_loop(0, B, edge, 0)
        pltpu.sync_copy(ex_v, expa_hbm.at[pl.ds(base0 + b * B, B)])
        pltpu.sync_copy(ex_v, acc.at[dst_v.at[pl.ds(b * B, B)]], add=True)

        @pl.when(b + NBUF < NB)
        def _():
            for cp in _gathers(b + NBUF, p):
                cp.start()
        return carry
    lax.fori_loop(0, NB, batch, 0)
    plsc.subcore_barrier()
    pltpu.sync_copy(acc.at[pl.ds(r0, R16)], denom_hbm.at[c].at[pl.ds(r0, R16)])


def _sc_attn(src, dst, asrc, adst, z16):
    return pl.kernel(
        _sc_attn_body,
        out_type=[jax.ShapeDtypeStruct((E_PAD, 16), jnp.float32),
                  jax.ShapeDtypeStruct((NC, PAD_N, 16), jnp.float32)],
        mesh=_MESH,
        compiler_params=_SC_PARAMS,
        scratch_types=[
            pltpu.VMEM((NBB,), jnp.int32),
            pltpu.VMEM((NBB,), jnp.int32),
            pltpu.VMEM((NBUF, B, 16), jnp.float32),
            pltpu.VMEM((NBUF, B, 16), jnp.float32),
            pltpu.VMEM((B, 16), jnp.float32),
            pltpu.VMEM_SHARED((PAD_N, 16), jnp.float32),
            pltpu.SemaphoreType.DMA((NBUF,)),
            pltpu.SemaphoreType.DMA((NBUF,)),
        ],
    )(src, dst, asrc, adst, z16)


def _sc_coef_body(dst_hbm, expa_hbm, denr_hbm, coef_hbm,
                  dst_v, ex_v, dr_v, cfT_v, sem1, sem2):
    c = lax.axis_index("c")
    s = lax.axis_index("s")
    wid = c * NS + s
    base0 = wid * NBB
    pltpu.sync_copy(dst_hbm.at[pl.ds(base0, NBB)], dst_v)
    lanes = lax.iota(jnp.int32, 16)

    def _loads(b, p):
        return (
            pltpu.make_async_copy(
                expa_hbm.at[pl.ds(base0 + b * B, B)], ex_v.at[p], sem1.at[p]),
            pltpu.make_async_copy(
                denr_hbm.at[dst_v.at[pl.ds(b * B, B)]], dr_v.at[p], sem2.at[p]),
        )

    for i in range(NBUF):
        for cp in _loads(i, i):
            cp.start()

    def batch(b, carry):
        p = lax.rem(b, NBUF)
        for cp in _loads(b, p):
            cp.wait()

        def edge(e, carry2):
            prod = ex_v[p, e] * dr_v[p, e]
            plsc.store_scatter(cfT_v, [lanes, jnp.full((16,), e, jnp.int32)],
                               prod)
            return carry2
        lax.fori_loop(0, B, edge, 0)
        pltpu.sync_copy(cfT_v.at[pl.ds(0, HEADS)],
                        coef_hbm.at[wid * NB + b])

        @pl.when(b + NBUF < NB)
        def _():
            for cp in _loads(b + NBUF, p):
                cp.start()
        return carry
    lax.fori_loop(0, NB, batch, 0)


def _sc_coef(dst, expa, denr):
    return pl.kernel(
        _sc_coef_body,
        out_type=[jax.ShapeDtypeStruct((NW * NB, HEADS, B), jnp.float32)],
        mesh=_MESH,
        compiler_params=_SC_PARAMS,
        scratch_types=[
            pltpu.VMEM((NBB,), jnp.int32),
            pltpu.VMEM((NBUF, B, 16), jnp.float32),
            pltpu.VMEM((NBUF, B, 16), jnp.float32),
            pltpu.VMEM((16, B), jnp.float32),
            pltpu.SemaphoreType.DMA((NBUF,)),
            pltpu.SemaphoreType.DMA((NBUF,)),
        ],
    )(dst, expa, denr)


def _make_agg_body(NCH):
    def body(src_hbm, dst_hbm, coefT_hbm, h8_hbm, zD_hbm, out_hbm,
             src_v, dst_v, cf_v, h_v, acc, sem1, sem2):
        c = lax.axis_index("c")
        s = lax.axis_index("s")
        wid = c * NS + s
        r0 = s * R16
        base0 = wid * NBB
        pltpu.sync_copy(src_hbm.at[pl.ds(base0, NBB)], src_v)
        pltpu.sync_copy(dst_hbm.at[pl.ds(base0, NBB)], dst_v)

        for k in range(NCH):
            def _loads(b, p, k=k):
                return (
                    pltpu.make_async_copy(
                        coefT_hbm.at[wid * NB + b].at[k], cf_v.at[p],
                        sem1.at[p]),
                    pltpu.make_async_copy(
                        h8_hbm.at[k].at[src_v.at[pl.ds(b * B, B)]], h_v.at[p],
                        sem2.at[p]),
                )

            for i in range(NBUF):
                for cp in _loads(i, i):
                    cp.start()
            pltpu.sync_copy(zD_hbm.at[pl.ds(r0, R16)], acc.at[pl.ds(r0, R16)])
            plsc.subcore_barrier()

            def batch(b, carry):
                p = lax.rem(b, NBUF)
                for cp in _loads(b, p):
                    cp.wait()

                def group(g, carry2):
                    cfvec = cf_v[p, pl.ds(g * 16, 16)]
                    for l in range(16):
                        cl = jnp.full((16,), cfvec[l], jnp.float32)
                        e = g * 16 + l
                        for j in range(HID // 16):
                            h_v[p, e, pl.ds(j * 16, 16)] = (
                                h_v[p, e, pl.ds(j * 16, 16)] * cl)
                    return carry2
                lax.fori_loop(0, B // 16, group, 0)
                pltpu.sync_copy(h_v.at[p],
                                acc.at[dst_v.at[pl.ds(b * B, B)]], add=True)

                @pl.when(b + NBUF < NB)
                def _():
                    for cp in _loads(b + NBUF, p):
                        cp.start()
                return carry
            lax.fori_loop(0, NB, batch, 0)
            plsc.subcore_barrier()
            pltpu.sync_copy(acc.at[pl.ds(r0, R16)],
                            out_hbm.at[k].at[c].at[pl.ds(r0, R16)])
    return body


def _sc_agg(src, dst, coefT, h8, zD, NCH):
    return pl.kernel(
        _make_agg_body(NCH),
        out_type=[jax.ShapeDtypeStruct((NCH, NC, PAD_N, HID), jnp.float32)],
        mesh=_MESH,
        compiler_params=_SC_PARAMS,
        scratch_types=[
            pltpu.VMEM((NBB,), jnp.int32),
            pltpu.VMEM((NBB,), jnp.int32),
            pltpu.VMEM((NBUF, B), jnp.float32),
            pltpu.VMEM((NBUF, B, HID), jnp.float32),
            pltpu.VMEM_SHARED((PAD_N, HID), jnp.float32),
            pltpu.SemaphoreType.DMA((NBUF,)),
            pltpu.SemaphoreType.DMA((NBUF,)),
        ],
    )(src, dst, coefT, h8, zD)


_ROWS = 256
_GRID = PAD_N // _ROWS


def _proj1_body(x_ref, w_ref, asp_ref, adp_ref, h_ref, as_ref, ad_ref):
    h = jnp.dot(x_ref[...], w_ref[...], preferred_element_type=jnp.float32)
    h_ref[...] = h
    as_ref[...] = jnp.dot(h, asp_ref[...], preferred_element_type=jnp.float32)
    ad_ref[...] = jnp.dot(h, adp_ref[...], preferred_element_type=jnp.float32)


def _proj1(x_pad, W1, As1p, Ad1p):
    D = W1.shape[1]
    return pl.pallas_call(
        _proj1_body,
        grid=(_GRID,),
        in_specs=[
            pl.BlockSpec((_ROWS, IN_CH), lambda i: (i, 0)),
            pl.BlockSpec((IN_CH, D), lambda i: (0, 0)),
            pl.BlockSpec((D, 16), lambda i: (0, 0)),
            pl.BlockSpec((D, 16), lambda i: (0, 0)),
        ],
        out_specs=[
            pl.BlockSpec((_ROWS, D), lambda i: (i, 0)),
            pl.BlockSpec((_ROWS, 16), lambda i: (i, 0)),
            pl.BlockSpec((_ROWS, 16), lambda i: (i, 0)),
        ],
        out_shape=[
            jax.ShapeDtypeStruct((PAD_N, D), jnp.float32),
            jax.ShapeDtypeStruct((PAD_N, 16), jnp.float32),
            jax.ShapeDtypeStruct((PAD_N, 16), jnp.float32),
        ],
    )(x_pad, W1, As1p, Ad1p)


def _recip_body(d_ref, o_ref):
    o_ref[...] = 1.0 / (d_ref[0] + d_ref[1] + 1e-16)


def _recip(denom):
    return pl.pallas_call(
        _recip_body,
        out_shape=jax.ShapeDtypeStruct((PAD_N, 16), jnp.float32),
    )(denom)


def _proj2_body(p0_ref, p1_ref, b1_ref, w2_ref, asp_ref, adp_ref,
                h2_ref, as_ref, ad_ref):
    z = p0_ref[...] + p1_ref[...] + b1_ref[...]
    z = jnp.maximum(z, 0.0) + jnp.exp(jnp.minimum(z, 0.0)) - 1.0
    h2 = jnp.dot(z, w2_ref[...], preferred_element_type=jnp.float32)
    h2_ref[...] = h2
    as_ref[...] = jnp.dot(h2, asp_ref[...], preferred_element_type=jnp.float32)
    ad_ref[...] = jnp.dot(h2, adp_ref[...], preferred_element_type=jnp.float32)


def _proj2(p0, p1, b1r, W2, As2p, Ad2p):
    D = p0.shape[1]
    return pl.pallas_call(
        _proj2_body,
        grid=(_GRID,),
        in_specs=[
            pl.BlockSpec((_ROWS, D), lambda i: (i, 0)),
            pl.BlockSpec((_ROWS, D), lambda i: (i, 0)),
            pl.BlockSpec((1, D), lambda i: (0, 0)),
            pl.BlockSpec((D, OUT_CH), lambda i: (0, 0)),
            pl.BlockSpec((OUT_CH, 16), lambda i: (0, 0)),
            pl.BlockSpec((OUT_CH, 16), lambda i: (0, 0)),
        ],
        out_specs=[
            pl.BlockSpec((_ROWS, OUT_CH), lambda i: (i, 0)),
            pl.BlockSpec((_ROWS, 16), lambda i: (i, 0)),
            pl.BlockSpec((_ROWS, 16), lambda i: (i, 0)),
        ],
        out_shape=[
            jax.ShapeDtypeStruct((PAD_N, OUT_CH), jnp.float32),
            jax.ShapeDtypeStruct((PAD_N, 16), jnp.float32),
            jax.ShapeDtypeStruct((PAD_N, 16), jnp.float32),
        ],
    )(p0, p1, b1r, W2, As2p, Ad2p)


def _final_body(p0_ref, p1_ref, b2_ref, o_ref):
    o_ref[...] = p0_ref[...] + p1_ref[...] + b2_ref[...]


def _final(p0, p1, b2r):
    return pl.pallas_call(
        _final_body,
        grid=(_GRID,),
        in_specs=[
            pl.BlockSpec((_ROWS, OUT_CH), lambda i: (i, 0)),
            pl.BlockSpec((_ROWS, OUT_CH), lambda i: (i, 0)),
            pl.BlockSpec((1, OUT_CH), lambda i: (0, 0)),
        ],
        out_specs=pl.BlockSpec((_ROWS, OUT_CH), lambda i: (i, 0)),
        out_shape=jax.ShapeDtypeStruct((PAD_N, OUT_CH), jnp.float32),
    )(p0, p1, b2r)


@jax.jit
def kernel(x, edge_index, W1, att_src1, att_dst1, bias1,
           W2, att_src2, att_dst2, bias2):
    ei = edge_index.astype(jnp.int32)
    loop = jnp.arange(N, dtype=jnp.int32)
    padlen = E_PAD - E_TOT
    pad = jnp.full((padlen,), N, jnp.int32)
    src = jnp.concatenate([ei[0], loop, pad])
    dst = jnp.concatenate([ei[1], loop, pad])

    D1 = HEADS * HID
    rows = jnp.arange(D1)
    cols = jnp.repeat(jnp.arange(HEADS), HID)
    As1p = jnp.zeros((D1, 16), jnp.float32).at[rows, cols].set(att_src1.reshape(-1))
    Ad1p = jnp.zeros((D1, 16), jnp.float32).at[rows, cols].set(att_dst1.reshape(-1))
    As2p = jnp.zeros((OUT_CH, 16), jnp.float32).at[:, 0].set(att_src2[0])
    Ad2p = jnp.zeros((OUT_CH, 16), jnp.float32).at[:, 0].set(att_dst2[0])

    x_pad = jnp.concatenate([x, jnp.zeros((PAD_N - N, IN_CH), jnp.float32)])
    z16 = jnp.zeros((PAD_N, 16), jnp.float32)
    z64 = jnp.zeros((PAD_N, OUT_CH), jnp.float32)

    h1, as1, ad1 = _proj1(x_pad, W1, As1p, Ad1p)
    expa1, denom1 = _sc_attn(src, dst, as1, ad1, z16)
    denr1 = _recip(denom1)
    (coefT1,) = _sc_coef(dst, expa1, denr1)
    h8 = h1.reshape(PAD_N, HEADS, HID).transpose(1, 0, 2)
    (agg1,) = _sc_agg(src, dst, coefT1, h8, z64, HEADS)
    p1cat = agg1.transpose(1, 2, 0, 3).reshape(NC, PAD_N, HEADS * HID)

    h2, as2, ad2 = _proj2(p1cat[0], p1cat[1], bias1.reshape(1, -1),
                          W2, As2p, Ad2p)
    expa2, denom2 = _sc_attn(src, dst, as2, ad2, z16)
    denr2 = _recip(denom2)
    (coefT2,) = _sc_coef(dst, expa2, denr2)
    (agg2,) = _sc_agg(src, dst, coefT2, h2.reshape(1, PAD_N, OUT_CH), z64, 1)
    out = _final(agg2[0, 0], agg2[0, 1], bias2.reshape(1, -1))
    return out[:N]


# speedup vs baseline: 1.3363x; 1.3363x over previous
import jax
import jax.numpy as jnp
from jax import lax
from jax.experimental import pallas as pl
from jax.experimental.pallas import tpu as pltpu
from jax.experimental.pallas import tpu_sc as plsc

N = 10000
IN_CH = 128
HID = 64
OUT_CH = 64
HEADS = 8
E_RAW = 320000
E_TOT = E_RAW + N
PAD_N = 10240
NC = 2
NS = 16
NW = NC * NS
B = 192
NB = 54
NBB = NB * B
E_PAD = NW * NBB
R16 = PAD_N // NS
NBUF = 4

_MESH = plsc.VectorSubcoreMesh(
    core_axis_name="c", subcore_axis_name="s", num_cores=NC, num_subcores=NS)
_SC_PARAMS = pltpu.CompilerParams(use_tc_tiling_on_sc=False)


def _sc_attn_body(src_hbm, dst_hbm, asrc_hbm, adst_hbm, z16_hbm,
                  expa_hbm, denom_hbm,
                  src_v, dst_v, rs_v, rd_v, ex_v, acc, sem1, sem2):
    c = lax.axis_index("c")
    s = lax.axis_index("s")
    wid = c * NS + s
    r0 = s * R16
    base0 = wid * NBB
    pltpu.sync_copy(src_hbm.at[pl.ds(base0, NBB)], src_v)
    pltpu.sync_copy(dst_hbm.at[pl.ds(base0, NBB)], dst_v)

    def _gathers(b, p):
        return (
            pltpu.make_async_copy(
                asrc_hbm.at[src_v.at[pl.ds(b * B, B)]], rs_v.at[p], sem1.at[p]),
            pltpu.make_async_copy(
                adst_hbm.at[dst_v.at[pl.ds(b * B, B)]], rd_v.at[p], sem2.at[p]),
        )

    for i in range(NBUF):
        for cp in _gathers(i, i):
            cp.start()
    pltpu.sync_copy(z16_hbm.at[pl.ds(r0, R16)], acc.at[pl.ds(r0, R16)])
    plsc.subcore_barrier()

    def batch(b, carry):
        p = lax.rem(b, NBUF)
        for cp in _gathers(b, p):
            cp.wait()

        def edge(e, carry2):
            a = rs_v[p, e] + rd_v[p, e]
            a = jnp.maximum(a, 0.2 * a)
            ex_v[e] = jnp.exp(a)
            return carry2
        lax.fori_loop(0, B, edge, 0)
        pltpu.sync_copy(ex_v, expa_hbm.at[pl.ds(base0 + b * B, B)])
        pltpu.sync_copy(ex_v, acc.at[dst_v.at[pl.ds(b * B, B)]], add=True)

        @pl.when(b + NBUF < NB)
        def _():
            for cp in _gathers(b + NBUF, p):
                cp.start()
        return carry
    lax.fori_loop(0, NB, batch, 0)
    plsc.subcore_barrier()
    pltpu.sync_copy(acc.at[pl.ds(r0, R16)], denom_hbm.at[c].at[pl.ds(r0, R16)])


def _sc_attn(src, dst, asrc, adst, z16):
    return pl.kernel(
        _sc_attn_body,
        out_type=[jax.ShapeDtypeStruct((E_PAD, 16), jnp.float32),
                  jax.ShapeDtypeStruct((NC, PAD_N, 16), jnp.float32)],
        mesh=_MESH,
        compiler_params=_SC_PARAMS,
        scratch_types=[
            pltpu.VMEM((NBB,), jnp.int32),
            pltpu.VMEM((NBB,), jnp.int32),
            pltpu.VMEM((NBUF, B, 16), jnp.float32),
            pltpu.VMEM((NBUF, B, 16), jnp.float32),
            pltpu.VMEM((B, 16), jnp.float32),
            pltpu.VMEM_SHARED((PAD_N, 16), jnp.float32),
            pltpu.SemaphoreType.DMA((NBUF,)),
            pltpu.SemaphoreType.DMA((NBUF,)),
        ],
    )(src, dst, asrc, adst, z16)


def _sc_coef_body(dst_hbm, expa_hbm, denr_hbm, coef_hbm,
                  dst_v, ex_v, dr_v, cf_v, sem1, sem2):
    c = lax.axis_index("c")
    s = lax.axis_index("s")
    wid = c * NS + s
    base0 = wid * NBB
    pltpu.sync_copy(dst_hbm.at[pl.ds(base0, NBB)], dst_v)

    def _loads(b, p):
        return (
            pltpu.make_async_copy(
                expa_hbm.at[pl.ds(base0 + b * B, B)], ex_v.at[p], sem1.at[p]),
            pltpu.make_async_copy(
                denr_hbm.at[dst_v.at[pl.ds(b * B, B)]], dr_v.at[p], sem2.at[p]),
        )

    for i in range(NBUF):
        for cp in _loads(i, i):
            cp.start()

    def batch(b, carry):
        p = lax.rem(b, NBUF)
        for cp in _loads(b, p):
            cp.wait()

        def edge(e, carry2):
            cf_v[e] = ex_v[p, e] * dr_v[p, e]
            return carry2
        lax.fori_loop(0, B, edge, 0)
        pltpu.sync_copy(cf_v, coef_hbm.at[pl.ds(base0 + b * B, B)])

        @pl.when(b + NBUF < NB)
        def _():
            for cp in _loads(b + NBUF, p):
                cp.start()
        return carry
    lax.fori_loop(0, NB, batch, 0)


def _sc_coef(dst, expa, denr):
    return pl.kernel(
        _sc_coef_body,
        out_type=[jax.ShapeDtypeStruct((E_PAD, 16), jnp.float32)],
        mesh=_MESH,
        compiler_params=_SC_PARAMS,
        scratch_types=[
            pltpu.VMEM((NBB,), jnp.int32),
            pltpu.VMEM((NBUF, B, 16), jnp.float32),
            pltpu.VMEM((NBUF, B, 16), jnp.float32),
            pltpu.VMEM((B, 16), jnp.float32),
            pltpu.SemaphoreType.DMA((NBUF,)),
            pltpu.SemaphoreType.DMA((NBUF,)),
        ],
    )(dst, expa, denr)


def _make_agg_body(Dc, la):
    NJ = Dc // 16

    def body(src_hbm, dst_hbm, coef1d_hbm, htab_hbm, zD_hbm, out_hbm,
             src_v, dst_v, cf_v, h_v, acc, sem1, sem2):
        c = lax.axis_index("c")
        s = lax.axis_index("s")
        wid = c * NS + s
        r0 = s * R16
        base0 = wid * NBB
        pltpu.sync_copy(src_hbm.at[pl.ds(base0, NBB)], src_v)
        pltpu.sync_copy(dst_hbm.at[pl.ds(base0, NBB)], dst_v)

        def _loads(b, p):
            return (
                pltpu.make_async_copy(
                    coef1d_hbm.at[pl.ds((base0 + b * B) * 16, B * 16)],
                    cf_v.at[p], sem1.at[p]),
                pltpu.make_async_copy(
                    htab_hbm.at[src_v.at[pl.ds(b * B, B)]], h_v.at[p],
                    sem2.at[p]),
            )

        for i in range(NBUF):
            for cp in _loads(i, i):
                cp.start()
        pltpu.sync_copy(zD_hbm.at[pl.ds(r0, R16)], acc.at[pl.ds(r0, R16)])
        plsc.subcore_barrier()

        def batch(b, carry):
            p = lax.rem(b, NBUF)
            for cp in _loads(b, p):
                cp.wait()

            def edge(e, carry2):
                cfrow = cf_v[p, pl.ds(e * 16, 16)]
                c0 = jnp.full((16,), cfrow[la], jnp.float32)
                for j in range(NJ):
                    h_v[p, e, pl.ds(j * 16, 16)] = h_v[p, e, pl.ds(j * 16, 16)] * c0
                return carry2
            lax.fori_loop(0, B, edge, 0)
            pltpu.sync_copy(h_v.at[p], acc.at[dst_v.at[pl.ds(b * B, B)]], add=True)

            @pl.when(b + NBUF < NB)
            def _():
                for cp in _loads(b + NBUF, p):
                    cp.start()
            return carry
        lax.fori_loop(0, NB, batch, 0)
        plsc.subcore_barrier()
        pltpu.sync_copy(acc.at[pl.ds(r0, R16)], out_hbm.at[c].at[pl.ds(r0, R16)])
    return body


def _sc_agg(src, dst, coef1d, htab, zD, Dc, la):
    return pl.kernel(
        _make_agg_body(Dc, la),
        out_type=[jax.ShapeDtypeStruct((NC, PAD_N, Dc), jnp.float32)],
        mesh=_MESH,
        compiler_params=_SC_PARAMS,
        scratch_types=[
            pltpu.VMEM((NBB,), jnp.int32),
            pltpu.VMEM((NBB,), jnp.int32),
            pltpu.VMEM((NBUF, B * 16), jnp.float32),
            pltpu.VMEM((NBUF, B, Dc), jnp.float32),
            pltpu.VMEM_SHARED((PAD_N, Dc), jnp.float32),
            pltpu.SemaphoreType.DMA((NBUF,)),
            pltpu.SemaphoreType.DMA((NBUF,)),
        ],
    )(src, dst, coef1d, htab, zD)


_ROWS = 256
_GRID = PAD_N // _ROWS


def _proj1_body(x_ref, w_ref, asp_ref, adp_ref, h_ref, as_ref, ad_ref):
    h = jnp.dot(x_ref[...], w_ref[...], preferred_element_type=jnp.float32)
    h_ref[...] = h
    as_ref[...] = jnp.dot(h, asp_ref[...], preferred_element_type=jnp.float32)
    ad_ref[...] = jnp.dot(h, adp_ref[...], preferred_element_type=jnp.float32)


def _proj1(x_pad, W1, As1p, Ad1p):
    D = W1.shape[1]
    return pl.pallas_call(
        _proj1_body,
        grid=(_GRID,),
        in_specs=[
            pl.BlockSpec((_ROWS, IN_CH), lambda i: (i, 0)),
            pl.BlockSpec((IN_CH, D), lambda i: (0, 0)),
            pl.BlockSpec((D, 16), lambda i: (0, 0)),
            pl.BlockSpec((D, 16), lambda i: (0, 0)),
        ],
        out_specs=[
            pl.BlockSpec((_ROWS, D), lambda i: (i, 0)),
            pl.BlockSpec((_ROWS, 16), lambda i: (i, 0)),
            pl.BlockSpec((_ROWS, 16), lambda i: (i, 0)),
        ],
        out_shape=[
            jax.ShapeDtypeStruct((PAD_N, D), jnp.float32),
            jax.ShapeDtypeStruct((PAD_N, 16), jnp.float32),
            jax.ShapeDtypeStruct((PAD_N, 16), jnp.float32),
        ],
    )(x_pad, W1, As1p, Ad1p)


def _recip_body(d_ref, o_ref):
    o_ref[...] = 1.0 / (d_ref[0] + d_ref[1] + 1e-16)


def _recip(denom):
    return pl.pallas_call(
        _recip_body,
        out_shape=jax.ShapeDtypeStruct((PAD_N, 16), jnp.float32),
    )(denom)


def _proj2_body(p0_ref, p1_ref, b1_ref, w2_ref, asp_ref, adp_ref,
                h2_ref, as_ref, ad_ref):
    z = p0_ref[...] + p1_ref[...] + b1_ref[...]
    z = jnp.maximum(z, 0.0) + jnp.exp(jnp.minimum(z, 0.0)) - 1.0
    h2 = jnp.dot(z, w2_ref[...], preferred_element_type=jnp.float32)
    h2_ref[...] = h2
    as_ref[...] = jnp.dot(h2, asp_ref[...], preferred_element_type=jnp.float32)
    ad_ref[...] = jnp.dot(h2, adp_ref[...], preferred_element_type=jnp.float32)


def _proj2(p0, p1, b1r, W2, As2p, Ad2p):
    D = p0.shape[1]
    return pl.pallas_call(
        _proj2_body,
        grid=(_GRID,),
        in_specs=[
            pl.BlockSpec((_ROWS, D), lambda i: (i, 0)),
            pl.BlockSpec((_ROWS, D), lambda i: (i, 0)),
            pl.BlockSpec((1, D), lambda i: (0, 0)),
            pl.BlockSpec((D, OUT_CH), lambda i: (0, 0)),
            pl.BlockSpec((OUT_CH, 16), lambda i: (0, 0)),
            pl.BlockSpec((OUT_CH, 16), lambda i: (0, 0)),
        ],
        out_specs=[
            pl.BlockSpec((_ROWS, OUT_CH), lambda i: (i, 0)),
            pl.BlockSpec((_ROWS, 16), lambda i: (i, 0)),
            pl.BlockSpec((_ROWS, 16), lambda i: (i, 0)),
        ],
        out_shape=[
            jax.ShapeDtypeStruct((PAD_N, OUT_CH), jnp.float32),
            jax.ShapeDtypeStruct((PAD_N, 16), jnp.float32),
            jax.ShapeDtypeStruct((PAD_N, 16), jnp.float32),
        ],
    )(p0, p1, b1r, W2, As2p, Ad2p)


def _final_body(p0_ref, p1_ref, b2_ref, o_ref):
    o_ref[...] = p0_ref[...] + p1_ref[...] + b2_ref[...]


def _final(p0, p1, b2r):
    return pl.pallas_call(
        _final_body,
        grid=(_GRID,),
        in_specs=[
            pl.BlockSpec((_ROWS, OUT_CH), lambda i: (i, 0)),
            pl.BlockSpec((_ROWS, OUT_CH), lambda i: (i, 0)),
            pl.BlockSpec((1, OUT_CH), lambda i: (0, 0)),
        ],
        out_specs=pl.BlockSpec((_ROWS, OUT_CH), lambda i: (i, 0)),
        out_shape=jax.ShapeDtypeStruct((PAD_N, OUT_CH), jnp.float32),
    )(p0, p1, b2r)


@jax.jit
def kernel(x, edge_index, W1, att_src1, att_dst1, bias1,
           W2, att_src2, att_dst2, bias2):
    ei = edge_index.astype(jnp.int32)
    loop = jnp.arange(N, dtype=jnp.int32)
    padlen = E_PAD - E_TOT
    pad = jnp.full((padlen,), N, jnp.int32)
    src = jnp.concatenate([ei[0], loop, pad])
    dst = jnp.concatenate([ei[1], loop, pad])

    D1 = HEADS * HID
    rows = jnp.arange(D1)
    cols = jnp.repeat(jnp.arange(HEADS), HID)
    As1p = jnp.zeros((D1, 16), jnp.float32).at[rows, cols].set(att_src1.reshape(-1))
    Ad1p = jnp.zeros((D1, 16), jnp.float32).at[rows, cols].set(att_dst1.reshape(-1))
    As2p = jnp.zeros((OUT_CH, 16), jnp.float32).at[:, 0].set(att_src2[0])
    Ad2p = jnp.zeros((OUT_CH, 16), jnp.float32).at[:, 0].set(att_dst2[0])

    x_pad = jnp.concatenate([x, jnp.zeros((PAD_N - N, IN_CH), jnp.float32)])
    z16 = jnp.zeros((PAD_N, 16), jnp.float32)
    z64 = jnp.zeros((PAD_N, OUT_CH), jnp.float32)

    h1, as1, ad1 = _proj1(x_pad, W1, As1p, Ad1p)
    expa1, denom1 = _sc_attn(src, dst, as1, ad1, z16)
    denr1 = _recip(denom1)
    (coef1,) = _sc_coef(dst, expa1, denr1)
    coef1d = coef1.reshape(-1)
    parts = []
    for k in range(HEADS):
        hk = lax.slice(h1, (0, HID * k), (PAD_N, HID * (k + 1)))
        (pk,) = _sc_agg(src, dst, coef1d, hk, z64, HID, k)
        parts.append(pk)
    p1cat = jnp.concatenate(parts, axis=2)

    h2, as2, ad2 = _proj2(p1cat[0], p1cat[1], bias1.reshape(1, -1),
                          W2, As2p, Ad2p)
    expa2, denom2 = _sc_attn(src, dst, as2, ad2, z16)
    denr2 = _recip(denom2)
    (coef2,) = _sc_coef(dst, expa2, denr2)
    (p2,) = _sc_agg(src, dst, coef2.reshape(-1), h2, z64, OUT_CH, 0)
    out = _final(p2[0], p2[1], bias2.reshape(1, -1))
    return out[:N]


# speedup vs baseline: 1.3376x; 1.0010x over previous
import jax
import jax.numpy as jnp
from jax import lax
from jax.experimental import pallas as pl
from jax.experimental.pallas import tpu as pltpu
from jax.experimental.pallas import tpu_sc as plsc

N = 10000
IN_CH = 128
HID = 64
OUT_CH = 64
HEADS = 8
E_RAW = 320000
E_TOT = E_RAW + N
PAD_N = 10240
NC = 2
NS = 16
NW = NC * NS
B = 192
NB = 54
NBB = NB * B
E_PAD = NW * NBB
R16 = PAD_N // NS
NBUF = 4

_MESH = plsc.VectorSubcoreMesh(
    core_axis_name="c", subcore_axis_name="s", num_cores=NC, num_subcores=NS)
_SC_PARAMS = pltpu.CompilerParams(use_tc_tiling_on_sc=False)


def _sc_attn_body(src_hbm, dst_hbm, asrc_hbm, adst_hbm, z16_hbm,
                  expa_hbm, denom_hbm,
                  src_v, dst_v, rs_v, rd_v, ex_v, acc, sem1, sem2):
    c = lax.axis_index("c")
    s = lax.axis_index("s")
    wid = c * NS + s
    r0 = s * R16
    base0 = wid * NBB
    pltpu.sync_copy(src_hbm.at[pl.ds(base0, NBB)], src_v)
    pltpu.sync_copy(dst_hbm.at[pl.ds(base0, NBB)], dst_v)

    def _gathers(b, p):
        return (
            pltpu.make_async_copy(
                asrc_hbm.at[src_v.at[pl.ds(b * B, B)]], rs_v.at[p], sem1.at[p]),
            pltpu.make_async_copy(
                adst_hbm.at[dst_v.at[pl.ds(b * B, B)]], rd_v.at[p], sem2.at[p]),
        )

    for i in range(NBUF):
        for cp in _gathers(i, i):
            cp.start()
    pltpu.sync_copy(z16_hbm.at[pl.ds(r0, R16)], acc.at[pl.ds(r0, R16)])
    plsc.subcore_barrier()

    def batch(b, carry):
        p = lax.rem(b, NBUF)
        for cp in _gathers(b, p):
            cp.wait()

        def edge(e, carry2):
            a = rs_v[p, e] + rd_v[p, e]
            a = jnp.maximum(a, 0.2 * a)
            ex_v[e] = jnp.exp(a)
            return carry2
        lax.fori_loop(0, B, edge, 0)
        pltpu.sync_copy(ex_v, expa_hbm.at[pl.ds(base0 + b * B, B)])
        pltpu.sync_copy(ex_v, acc.at[dst_v.at[pl.ds(b * B, B)]], add=True)

        @pl.when(b + NBUF < NB)
        def _():
            for cp in _gathers(b + NBUF, p):
                cp.start()
        return carry
    lax.fori_loop(0, NB, batch, 0)
    plsc.subcore_barrier()
    pltpu.sync_copy(acc.at[pl.ds(r0, R16)], denom_hbm.at[c].at[pl.ds(r0, R16)])


def _sc_attn(src, dst, asrc, adst, z16):
    return pl.kernel(
        _sc_attn_body,
        out_type=[jax.ShapeDtypeStruct((E_PAD, 16), jnp.float32),
                  jax.ShapeDtypeStruct((NC, PAD_N, 16), jnp.float32)],
        mesh=_MESH,
        compiler_params=_SC_PARAMS,
        scratch_types=[
            pltpu.VMEM((NBB,), jnp.int32),
            pltpu.VMEM((NBB,), jnp.int32),
            pltpu.VMEM((NBUF, B, 16), jnp.float32),
            pltpu.VMEM((NBUF, B, 16), jnp.float32),
            pltpu.VMEM((B, 16), jnp.float32),
            pltpu.VMEM_SHARED((PAD_N, 16), jnp.float32),
            pltpu.SemaphoreType.DMA((NBUF,)),
            pltpu.SemaphoreType.DMA((NBUF,)),
        ],
    )(src, dst, asrc, adst, z16)


def _sc_coef_body(dst_hbm, expa_hbm, denr_hbm, coef_hbm,
                  dst_v, ex_v, dr_v, cf_v, sem1, sem2):
    c = lax.axis_index("c")
    s = lax.axis_index("s")
    wid = c * NS + s
    base0 = wid * NBB
    pltpu.sync_copy(dst_hbm.at[pl.ds(base0, NBB)], dst_v)

    def _loads(b, p):
        return (
            pltpu.make_async_copy(
                expa_hbm.at[pl.ds(base0 + b * B, B)], ex_v.at[p], sem1.at[p]),
            pltpu.make_async_copy(
                denr_hbm.at[dst_v.at[pl.ds(b * B, B)]], dr_v.at[p], sem2.at[p]),
        )

    for i in range(NBUF):
        for cp in _loads(i, i):
            cp.start()

    def batch(b, carry):
        p = lax.rem(b, NBUF)
        for cp in _loads(b, p):
            cp.wait()

        def edge(e, carry2):
            cf_v[e] = ex_v[p, e] * dr_v[p, e]
            return carry2
        lax.fori_loop(0, B, edge, 0)
        pltpu.sync_copy(cf_v, coef_hbm.at[pl.ds(base0 + b * B, B)])

        @pl.when(b + NBUF < NB)
        def _():
            for cp in _loads(b + NBUF, p):
                cp.start()
        return carry
    lax.fori_loop(0, NB, batch, 0)


def _sc_coef(dst, expa, denr):
    return pl.kernel(
        _sc_coef_body,
        out_type=[jax.ShapeDtypeStruct((E_PAD, 16), jnp.float32)],
        mesh=_MESH,
        compiler_params=_SC_PARAMS,
        scratch_types=[
            pltpu.VMEM((NBB,), jnp.int32),
            pltpu.VMEM((NBUF, B, 16), jnp.float32),
            pltpu.VMEM((NBUF, B, 16), jnp.float32),
            pltpu.VMEM((B, 16), jnp.float32),
            pltpu.SemaphoreType.DMA((NBUF,)),
            pltpu.SemaphoreType.DMA((NBUF,)),
        ],
    )(dst, expa, denr)


def _make_agg_body(Dc, la):
    NJ = Dc // 16

    def body(src_hbm, dst_hbm, coef1d_hbm, htab_hbm, zD_hbm, out_hbm,
             src_v, dst_v, cf_v, h_v, acc, sem1, sem2, sem3):
        c = lax.axis_index("c")
        s = lax.axis_index("s")
        wid = c * NS + s
        r0 = s * R16
        base0 = wid * NBB
        pltpu.sync_copy(src_hbm.at[pl.ds(base0, NBB)], src_v)
        pltpu.sync_copy(dst_hbm.at[pl.ds(base0, NBB)], dst_v)

        def _loads(b, p):
            return (
                pltpu.make_async_copy(
                    coef1d_hbm.at[pl.ds((base0 + b * B) * 16, B * 16)],
                    cf_v.at[p], sem1.at[p]),
                pltpu.make_async_copy(
                    htab_hbm.at[src_v.at[pl.ds(b * B, B)]], h_v.at[p],
                    sem2.at[p]),
            )

        for i in range(NBUF):
            for cp in _loads(i, i):
                cp.start()
        pltpu.sync_copy(zD_hbm.at[pl.ds(r0, R16)], acc.at[pl.ds(r0, R16)])
        plsc.subcore_barrier()

        def batch(b, carry):
            p = lax.rem(b, NBUF)

            @pl.when(b > 0)
            def _():
                bp = b - 1
                pp = lax.rem(bp, NBUF)
                pltpu.make_async_copy(
                    h_v.at[pp], acc.at[dst_v.at[pl.ds(bp * B, B)]],
                    sem3.at[pp]).wait()

                @pl.when(bp + NBUF < NB)
                def _():
                    for cp in _loads(bp + NBUF, pp):
                        cp.start()
            for cp in _loads(b, p):
                cp.wait()

            def edge(e, carry2):
                cfrow = cf_v[p, pl.ds(e * 16, 16)]
                c0 = jnp.full((16,), cfrow[la], jnp.float32)
                for j in range(NJ):
                    h_v[p, e, pl.ds(j * 16, 16)] = h_v[p, e, pl.ds(j * 16, 16)] * c0
                return carry2
            lax.fori_loop(0, B, edge, 0)
            pltpu.async_copy(h_v.at[p], acc.at[dst_v.at[pl.ds(b * B, B)]],
                             sem3.at[p], add=True)
            return carry
        lax.fori_loop(0, NB, batch, 0)
        bl = NB - 1
        pltpu.make_async_copy(
            h_v.at[lax.rem(bl, NBUF)],
            acc.at[dst_v.at[pl.ds(bl * B, B)]],
            sem3.at[lax.rem(bl, NBUF)]).wait()
        plsc.subcore_barrier()
        pltpu.sync_copy(acc.at[pl.ds(r0, R16)], out_hbm.at[c].at[pl.ds(r0, R16)])
    return body


def _sc_agg(src, dst, coef1d, htab, zD, Dc, la):
    return pl.kernel(
        _make_agg_body(Dc, la),
        out_type=[jax.ShapeDtypeStruct((NC, PAD_N, Dc), jnp.float32)],
        mesh=_MESH,
        compiler_params=_SC_PARAMS,
        scratch_types=[
            pltpu.VMEM((NBB,), jnp.int32),
            pltpu.VMEM((NBB,), jnp.int32),
            pltpu.VMEM((NBUF, B * 16), jnp.float32),
            pltpu.VMEM((NBUF, B, Dc), jnp.float32),
            pltpu.VMEM_SHARED((PAD_N, Dc), jnp.float32),
            pltpu.SemaphoreType.DMA((NBUF,)),
            pltpu.SemaphoreType.DMA((NBUF,)),
            pltpu.SemaphoreType.DMA((NBUF,)),
        ],
    )(src, dst, coef1d, htab, zD)


_ROWS = 256
_GRID = PAD_N // _ROWS


def _proj1_body(x_ref, w_ref, asp_ref, adp_ref, h_ref, as_ref, ad_ref):
    h = jnp.dot(x_ref[...], w_ref[...], preferred_element_type=jnp.float32)
    h_ref[...] = h
    as_ref[...] = jnp.dot(h, asp_ref[...], preferred_element_type=jnp.float32)
    ad_ref[...] = jnp.dot(h, adp_ref[...], preferred_element_type=jnp.float32)


def _proj1(x_pad, W1, As1p, Ad1p):
    D = W1.shape[1]
    return pl.pallas_call(
        _proj1_body,
        grid=(_GRID,),
        in_specs=[
            pl.BlockSpec((_ROWS, IN_CH), lambda i: (i, 0)),
            pl.BlockSpec((IN_CH, D), lambda i: (0, 0)),
            pl.BlockSpec((D, 16), lambda i: (0, 0)),
            pl.BlockSpec((D, 16), lambda i: (0, 0)),
        ],
        out_specs=[
            pl.BlockSpec((_ROWS, D), lambda i: (i, 0)),
            pl.BlockSpec((_ROWS, 16), lambda i: (i, 0)),
            pl.BlockSpec((_ROWS, 16), lambda i: (i, 0)),
        ],
        out_shape=[
            jax.ShapeDtypeStruct((PAD_N, D), jnp.float32),
            jax.ShapeDtypeStruct((PAD_N, 16), jnp.float32),
            jax.ShapeDtypeStruct((PAD_N, 16), jnp.float32),
        ],
    )(x_pad, W1, As1p, Ad1p)


def _recip_body(d_ref, o_ref):
    o_ref[...] = 1.0 / (d_ref[0] + d_ref[1] + 1e-16)


def _recip(denom):
    return pl.pallas_call(
        _recip_body,
        out_shape=jax.ShapeDtypeStruct((PAD_N, 16), jnp.float32),
    )(denom)


def _proj2_body(p0_ref, p1_ref, b1_ref, w2_ref, asp_ref, adp_ref,
                h2_ref, as_ref, ad_ref):
    z = p0_ref[...] + p1_ref[...] + b1_ref[...]
    z = jnp.maximum(z, 0.0) + jnp.exp(jnp.minimum(z, 0.0)) - 1.0
    h2 = jnp.dot(z, w2_ref[...], preferred_element_type=jnp.float32)
    h2_ref[...] = h2
    as_ref[...] = jnp.dot(h2, asp_ref[...], preferred_element_type=jnp.float32)
    ad_ref[...] = jnp.dot(h2, adp_ref[...], preferred_element_type=jnp.float32)


def _proj2(p0, p1, b1r, W2, As2p, Ad2p):
    D = p0.shape[1]
    return pl.pallas_call(
        _proj2_body,
        grid=(_GRID,),
        in_specs=[
            pl.BlockSpec((_ROWS, D), lambda i: (i, 0)),
            pl.BlockSpec((_ROWS, D), lambda i: (i, 0)),
            pl.BlockSpec((1, D), lambda i: (0, 0)),
            pl.BlockSpec((D, OUT_CH), lambda i: (0, 0)),
            pl.BlockSpec((OUT_CH, 16), lambda i: (0, 0)),
            pl.BlockSpec((OUT_CH, 16), lambda i: (0, 0)),
        ],
        out_specs=[
            pl.BlockSpec((_ROWS, OUT_CH), lambda i: (i, 0)),
            pl.BlockSpec((_ROWS, 16), lambda i: (i, 0)),
            pl.BlockSpec((_ROWS, 16), lambda i: (i, 0)),
        ],
        out_shape=[
            jax.ShapeDtypeStruct((PAD_N, OUT_CH), jnp.float32),
            jax.ShapeDtypeStruct((PAD_N, 16), jnp.float32),
            jax.ShapeDtypeStruct((PAD_N, 16), jnp.float32),
        ],
    )(p0, p1, b1r, W2, As2p, Ad2p)


def _final_body(p0_ref, p1_ref, b2_ref, o_ref):
    o_ref[...] = p0_ref[...] + p1_ref[...] + b2_ref[...]


def _final(p0, p1, b2r):
    return pl.pallas_call(
        _final_body,
        grid=(_GRID,),
        in_specs=[
            pl.BlockSpec((_ROWS, OUT_CH), lambda i: (i, 0)),
            pl.BlockSpec((_ROWS, OUT_CH), lambda i: (i, 0)),
            pl.BlockSpec((1, OUT_CH), lambda i: (0, 0)),
        ],
        out_specs=pl.BlockSpec((_ROWS, OUT_CH), lambda i: (i, 0)),
        out_shape=jax.ShapeDtypeStruct((PAD_N, OUT_CH), jnp.float32),
    )(p0, p1, b2r)


@jax.jit
def kernel(x, edge_index, W1, att_src1, att_dst1, bias1,
           W2, att_src2, att_dst2, bias2):
    ei = edge_index.astype(jnp.int32)
    loop = jnp.arange(N, dtype=jnp.int32)
    padlen = E_PAD - E_TOT
    pad = jnp.full((padlen,), N, jnp.int32)
    src = jnp.concatenate([ei[0], loop, pad])
    dst = jnp.concatenate([ei[1], loop, pad])

    D1 = HEADS * HID
    rows = jnp.arange(D1)
    cols = jnp.repeat(jnp.arange(HEADS), HID)
    As1p = jnp.zeros((D1, 16), jnp.float32).at[rows, cols].set(att_src1.reshape(-1))
    Ad1p = jnp.zeros((D1, 16), jnp.float32).at[rows, cols].set(att_dst1.reshape(-1))
    As2p = jnp.zeros((OUT_CH, 16), jnp.float32).at[:, 0].set(att_src2[0])
    Ad2p = jnp.zeros((OUT_CH, 16), jnp.float32).at[:, 0].set(att_dst2[0])

    x_pad = jnp.concatenate([x, jnp.zeros((PAD_N - N, IN_CH), jnp.float32)])
    z16 = jnp.zeros((PAD_N, 16), jnp.float32)
    z64 = jnp.zeros((PAD_N, OUT_CH), jnp.float32)

    h1, as1, ad1 = _proj1(x_pad, W1, As1p, Ad1p)
    expa1, denom1 = _sc_attn(src, dst, as1, ad1, z16)
    denr1 = _recip(denom1)
    (coef1,) = _sc_coef(dst, expa1, denr1)
    coef1d = coef1.reshape(-1)
    parts = []
    for k in range(HEADS):
        hk = lax.slice(h1, (0, HID * k), (PAD_N, HID * (k + 1)))
        (pk,) = _sc_agg(src, dst, coef1d, hk, z64, HID, k)
        parts.append(pk)
    p1cat = jnp.concatenate(parts, axis=2)

    h2, as2, ad2 = _proj2(p1cat[0], p1cat[1], bias1.reshape(1, -1),
                          W2, As2p, Ad2p)
    expa2, denom2 = _sc_attn(src, dst, as2, ad2, z16)
    denr2 = _recip(denom2)
    (coef2,) = _sc_coef(dst, expa2, denr2)
    (p2,) = _sc_agg(src, dst, coef2.reshape(-1), h2, z64, OUT_CH, 0)
    out = _final(p2[0], p2[1], bias2.reshape(1, -1))
    return out[:N]


# speedup vs baseline: 1.3428x; 1.0039x over previous
import jax
import jax.numpy as jnp
from jax import lax
from jax.experimental import pallas as pl
from jax.experimental.pallas import tpu as pltpu
from jax.experimental.pallas import tpu_sc as plsc

N = 10000
IN_CH = 128
HID = 64
OUT_CH = 64
HEADS = 8
E_RAW = 320000
E_TOT = E_RAW + N
PAD_N = 10240
NC = 2
NS = 16
NW = NC * NS
B = 192
NB = 54
NBB = NB * B
E_PAD = NW * NBB
R16 = PAD_N // NS
NBUF = 4

_MESH = plsc.VectorSubcoreMesh(
    core_axis_name="c", subcore_axis_name="s", num_cores=NC, num_subcores=NS)
_SC_PARAMS = pltpu.CompilerParams(use_tc_tiling_on_sc=False)


def _sc_attn_body(src_hbm, dst_hbm, asrc_hbm, adst_hbm, z16_hbm,
                  expa_hbm, denom_hbm,
                  src_v, dst_v, rs_v, rd_v, ex_v, acc, sem1, sem2):
    c = lax.axis_index("c")
    s = lax.axis_index("s")
    wid = c * NS + s
    r0 = s * R16
    base0 = wid * NBB
    pltpu.sync_copy(src_hbm.at[pl.ds(base0, NBB)], src_v)
    pltpu.sync_copy(dst_hbm.at[pl.ds(base0, NBB)], dst_v)

    def _gathers(b, p):
        return (
            pltpu.make_async_copy(
                asrc_hbm.at[src_v.at[pl.ds(b * B, B)]], rs_v.at[p], sem1.at[p]),
            pltpu.make_async_copy(
                adst_hbm.at[dst_v.at[pl.ds(b * B, B)]], rd_v.at[p], sem2.at[p]),
        )

    for i in range(NBUF):
        for cp in _gathers(i, i):
            cp.start()
    pltpu.sync_copy(z16_hbm.at[pl.ds(r0, R16)], acc.at[pl.ds(r0, R16)])
    plsc.subcore_barrier()

    def batch(b, carry):
        p = lax.rem(b, NBUF)
        for cp in _gathers(b, p):
            cp.wait()

        def edge(e, carry2):
            a = rs_v[p, e] + rd_v[p, e]
            a = jnp.maximum(a, 0.2 * a)
            ex_v[e] = jnp.exp(a)
            return carry2
        lax.fori_loop(0, B, edge, 0)
        pltpu.sync_copy(ex_v, expa_hbm.at[pl.ds(base0 + b * B, B)])
        pltpu.sync_copy(ex_v, acc.at[dst_v.at[pl.ds(b * B, B)]], add=True)

        @pl.when(b + NBUF < NB)
        def _():
            for cp in _gathers(b + NBUF, p):
                cp.start()
        return carry
    lax.fori_loop(0, NB, batch, 0)
    plsc.subcore_barrier()
    pltpu.sync_copy(acc.at[pl.ds(r0, R16)], denom_hbm.at[c].at[pl.ds(r0, R16)])


def _sc_attn(src, dst, asrc, adst, z16):
    return pl.kernel(
        _sc_attn_body,
        out_type=[jax.ShapeDtypeStruct((E_PAD, 16), jnp.float32),
                  jax.ShapeDtypeStruct((NC, PAD_N, 16), jnp.float32)],
        mesh=_MESH,
        compiler_params=_SC_PARAMS,
        scratch_types=[
            pltpu.VMEM((NBB,), jnp.int32),
            pltpu.VMEM((NBB,), jnp.int32),
            pltpu.VMEM((NBUF, B, 16), jnp.float32),
            pltpu.VMEM((NBUF, B, 16), jnp.float32),
            pltpu.VMEM((B, 16), jnp.float32),
            pltpu.VMEM_SHARED((PAD_N, 16), jnp.float32),
            pltpu.SemaphoreType.DMA((NBUF,)),
            pltpu.SemaphoreType.DMA((NBUF,)),
        ],
    )(src, dst, asrc, adst, z16)


def _sc_coef_body(dst_hbm, expa_hbm, denr_hbm, coef_hbm,
                  dst_v, ex_v, dr_v, cf_v, sem1, sem2):
    c = lax.axis_index("c")
    s = lax.axis_index("s")
    wid = c * NS + s
    base0 = wid * NBB
    pltpu.sync_copy(dst_hbm.at[pl.ds(base0, NBB)], dst_v)

    def _loads(b, p):
        return (
            pltpu.make_async_copy(
                expa_hbm.at[pl.ds(base0 + b * B, B)], ex_v.at[p], sem1.at[p]),
            pltpu.make_async_copy(
                denr_hbm.at[dst_v.at[pl.ds(b * B, B)]], dr_v.at[p], sem2.at[p]),
        )

    for i in range(NBUF):
        for cp in _loads(i, i):
            cp.start()

    def batch(b, carry):
        p = lax.rem(b, NBUF)
        for cp in _loads(b, p):
            cp.wait()

        def edge(e, carry2):
            cf_v[e] = ex_v[p, e] * dr_v[p, e]
            return carry2
        lax.fori_loop(0, B, edge, 0)
        pltpu.sync_copy(cf_v, coef_hbm.at[pl.ds(base0 + b * B, B)])

        @pl.when(b + NBUF < NB)
        def _():
            for cp in _loads(b + NBUF, p):
                cp.start()
        return carry
    lax.fori_loop(0, NB, batch, 0)


def _sc_coef(dst, expa, denr):
    return pl.kernel(
        _sc_coef_body,
        out_type=[jax.ShapeDtypeStruct((E_PAD, 16), jnp.float32)],
        mesh=_MESH,
        compiler_params=_SC_PARAMS,
        scratch_types=[
            pltpu.VMEM((NBB,), jnp.int32),
            pltpu.VMEM((NBUF, B, 16), jnp.float32),
            pltpu.VMEM((NBUF, B, 16), jnp.float32),
            pltpu.VMEM((B, 16), jnp.float32),
            pltpu.SemaphoreType.DMA((NBUF,)),
            pltpu.SemaphoreType.DMA((NBUF,)),
        ],
    )(dst, expa, denr)


def _make_agg_body(Dc, la):
    NJ = Dc // 16

    def body(src_hbm, dst_hbm, coef1d_hbm, htab_hbm, zD_hbm, out_hbm,
             src_v, dst_v, cf_v, h_v, acc, sem1, sem2, sem3):
        c = lax.axis_index("c")
        s = lax.axis_index("s")
        wid = c * NS + s
        r0 = s * R16
        base0 = wid * NBB
        pltpu.sync_copy(src_hbm.at[pl.ds(base0, NBB)], src_v)
        pltpu.sync_copy(dst_hbm.at[pl.ds(base0, NBB)], dst_v)

        def _loads(b, p):
            return (
                pltpu.make_async_copy(
                    coef1d_hbm.at[pl.ds((base0 + b * B) * 16, B * 16)],
                    cf_v.at[p], sem1.at[p]),
                pltpu.make_async_copy(
                    htab_hbm.at[src_v.at[pl.ds(b * B, B)]], h_v.at[p],
                    sem2.at[p]),
            )

        for i in range(NBUF):
            for cp in _loads(i, i):
                cp.start()
        pltpu.sync_copy(zD_hbm.at[pl.ds(r0, R16)], acc.at[pl.ds(r0, R16)])
        plsc.subcore_barrier()

        def batch(b, carry):
            p = lax.rem(b, NBUF)

            @pl.when(b > 0)
            def _():
                bp = b - 1
                pp = lax.rem(bp, NBUF)
                pltpu.make_async_copy(
                    h_v.at[pp], acc.at[dst_v.at[pl.ds(bp * B, B)]],
                    sem3.at[pp]).wait()

                @pl.when(bp + NBUF < NB)
                def _():
                    for cp in _loads(bp + NBUF, pp):
                        cp.start()
            for cp in _loads(b, p):
                cp.wait()

            def edge(e, carry2):
                cfrow = cf_v[p, pl.ds(e * 16, 16)]
                c0 = jnp.full((16,), cfrow[la], jnp.float32)
                for j in range(NJ):
                    h_v[p, e, pl.ds(j * 16, 16)] = h_v[p, e, pl.ds(j * 16, 16)] * c0
                return carry2
            lax.fori_loop(0, B, edge, 0)
            pltpu.async_copy(h_v.at[p], acc.at[dst_v.at[pl.ds(b * B, B)]],
                             sem3.at[p], add=True)
            return carry
        lax.fori_loop(0, NB, batch, 0)
        bl = NB - 1
        pltpu.make_async_copy(
            h_v.at[lax.rem(bl, NBUF)],
            acc.at[dst_v.at[pl.ds(bl * B, B)]],
            sem3.at[lax.rem(bl, NBUF)]).wait()
        plsc.subcore_barrier()
        pltpu.sync_copy(acc.at[pl.ds(r0, R16)], out_hbm.at[c].at[pl.ds(r0, R16)])
    return body


def _sc_agg(src, dst, coef1d, htab, zD, Dc, la):
    return pl.kernel(
        _make_agg_body(Dc, la),
        out_type=[jax.ShapeDtypeStruct((NC, PAD_N, Dc), jnp.float32)],
        mesh=_MESH,
        compiler_params=_SC_PARAMS,
        scratch_types=[
            pltpu.VMEM((NBB,), jnp.int32),
            pltpu.VMEM((NBB,), jnp.int32),
            pltpu.VMEM((NBUF, B * 16), jnp.float32),
            pltpu.VMEM((NBUF, B, Dc), jnp.float32),
            pltpu.VMEM_SHARED((PAD_N, Dc), jnp.float32),
            pltpu.SemaphoreType.DMA((NBUF,)),
            pltpu.SemaphoreType.DMA((NBUF,)),
            pltpu.SemaphoreType.DMA((NBUF,)),
        ],
    )(src, dst, coef1d, htab, zD)


def _agg8_body(src_hbm, dst_hbm, coef1d_hbm,
               h0, h1, h2, h3, h4, h5, h6, h7, zD_hbm, out_hbm,
               src_v, dst_v, cf_v, h_v, acc, sem1, sem2, sem3):
    c = lax.axis_index("c")
    s = lax.axis_index("s")
    wid = c * NS + s
    r0 = s * R16
    base0 = wid * NBB
    pltpu.sync_copy(src_hbm.at[pl.ds(base0, NBB)], src_v)
    pltpu.sync_copy(dst_hbm.at[pl.ds(base0, NBB)], dst_v)
    htabs = (h0, h1, h2, h3, h4, h5, h6, h7)

    for k in range(HEADS):
        htab_hbm = htabs[k]

        def _loads(b, p, htab_hbm=htab_hbm):
            return (
                pltpu.make_async_copy(
                    coef1d_hbm.at[pl.ds((base0 + b * B) * 16, B * 16)],
                    cf_v.at[p], sem1.at[p]),
                pltpu.make_async_copy(
                    htab_hbm.at[src_v.at[pl.ds(b * B, B)]], h_v.at[p],
                    sem2.at[p]),
            )

        for i in range(NBUF):
            for cp in _loads(i, i):
                cp.start()
        pltpu.sync_copy(zD_hbm.at[pl.ds(r0, R16)], acc.at[pl.ds(r0, R16)])
        plsc.subcore_barrier()

        def batch(b, carry, _loads=_loads, la=k):
            p = lax.rem(b, NBUF)

            @pl.when(b > 0)
            def _():
                bp = b - 1
                pp = lax.rem(bp, NBUF)
                pltpu.make_async_copy(
                    h_v.at[pp], acc.at[dst_v.at[pl.ds(bp * B, B)]],
                    sem3.at[pp]).wait()

                @pl.when(bp + NBUF < NB)
                def _():
                    for cp in _loads(bp + NBUF, pp):
                        cp.start()
            for cp in _loads(b, p):
                cp.wait()

            def edge(e, carry2):
                cfrow = cf_v[p, pl.ds(e * 16, 16)]
                c0 = jnp.full((16,), cfrow[la], jnp.float32)
                for j in range(HID // 16):
                    h_v[p, e, pl.ds(j * 16, 16)] = h_v[p, e, pl.ds(j * 16, 16)] * c0
                return carry2
            lax.fori_loop(0, B, edge, 0)
            pltpu.async_copy(h_v.at[p], acc.at[dst_v.at[pl.ds(b * B, B)]],
                             sem3.at[p], add=True)
            return carry
        lax.fori_loop(0, NB, batch, 0)
        bl = NB - 1
        pltpu.make_async_copy(
            h_v.at[lax.rem(bl, NBUF)],
            acc.at[dst_v.at[pl.ds(bl * B, B)]],
            sem3.at[lax.rem(bl, NBUF)]).wait()
        plsc.subcore_barrier()
        pltpu.sync_copy(acc.at[pl.ds(r0, R16)],
                        out_hbm.at[k].at[c].at[pl.ds(r0, R16)])


def _sc_agg8(src, dst, coef1d, hks, zD):
    return pl.kernel(
        _agg8_body,
        out_type=[jax.ShapeDtypeStruct((HEADS, NC, PAD_N, HID), jnp.float32)],
        mesh=_MESH,
        compiler_params=_SC_PARAMS,
        scratch_types=[
            pltpu.VMEM((NBB,), jnp.int32),
            pltpu.VMEM((NBB,), jnp.int32),
            pltpu.VMEM((NBUF, B * 16), jnp.float32),
            pltpu.VMEM((NBUF, B, HID), jnp.float32),
            pltpu.VMEM_SHARED((PAD_N, HID), jnp.float32),
            pltpu.SemaphoreType.DMA((NBUF,)),
            pltpu.SemaphoreType.DMA((NBUF,)),
            pltpu.SemaphoreType.DMA((NBUF,)),
        ],
    )(src, dst, coef1d, *hks, zD)


_ROWS = 256
_GRID = PAD_N // _ROWS


def _proj1_body(x_ref, w_ref, asp_ref, adp_ref, h_ref, as_ref, ad_ref):
    h = jnp.dot(x_ref[...], w_ref[...], preferred_element_type=jnp.float32)
    h_ref[...] = h
    as_ref[...] = jnp.dot(h, asp_ref[...], preferred_element_type=jnp.float32)
    ad_ref[...] = jnp.dot(h, adp_ref[...], preferred_element_type=jnp.float32)


def _proj1(x_pad, W1, As1p, Ad1p):
    D = W1.shape[1]
    return pl.pallas_call(
        _proj1_body,
        grid=(_GRID,),
        in_specs=[
            pl.BlockSpec((_ROWS, IN_CH), lambda i: (i, 0)),
            pl.BlockSpec((IN_CH, D), lambda i: (0, 0)),
            pl.BlockSpec((D, 16), lambda i: (0, 0)),
            pl.BlockSpec((D, 16), lambda i: (0, 0)),
        ],
        out_specs=[
            pl.BlockSpec((_ROWS, D), lambda i: (i, 0)),
            pl.BlockSpec((_ROWS, 16), lambda i: (i, 0)),
            pl.BlockSpec((_ROWS, 16), lambda i: (i, 0)),
        ],
        out_shape=[
            jax.ShapeDtypeStruct((PAD_N, D), jnp.float32),
            jax.ShapeDtypeStruct((PAD_N, 16), jnp.float32),
            jax.ShapeDtypeStruct((PAD_N, 16), jnp.float32),
        ],
    )(x_pad, W1, As1p, Ad1p)


def _recip_body(d_ref, o_ref):
    o_ref[...] = 1.0 / (d_ref[0] + d_ref[1] + 1e-16)


def _recip(denom):
    return pl.pallas_call(
        _recip_body,
        out_shape=jax.ShapeDtypeStruct((PAD_N, 16), jnp.float32),
    )(denom)


def _proj2_body(p0_ref, p1_ref, b1_ref, w2_ref, asp_ref, adp_ref,
                h2_ref, as_ref, ad_ref):
    z = p0_ref[...] + p1_ref[...] + b1_ref[...]
    z = jnp.maximum(z, 0.0) + jnp.exp(jnp.minimum(z, 0.0)) - 1.0
    h2 = jnp.dot(z, w2_ref[...], preferred_element_type=jnp.float32)
    h2_ref[...] = h2
    as_ref[...] = jnp.dot(h2, asp_ref[...], preferred_element_type=jnp.float32)
    ad_ref[...] = jnp.dot(h2, adp_ref[...], preferred_element_type=jnp.float32)


def _proj2(p0, p1, b1r, W2, As2p, Ad2p):
    D = p0.shape[1]
    return pl.pallas_call(
        _proj2_body,
        grid=(_GRID,),
        in_specs=[
            pl.BlockSpec((_ROWS, D), lambda i: (i, 0)),
            pl.BlockSpec((_ROWS, D), lambda i: (i, 0)),
            pl.BlockSpec((1, D), lambda i: (0, 0)),
            pl.BlockSpec((D, OUT_CH), lambda i: (0, 0)),
            pl.BlockSpec((OUT_CH, 16), lambda i: (0, 0)),
            pl.BlockSpec((OUT_CH, 16), lambda i: (0, 0)),
        ],
        out_specs=[
            pl.BlockSpec((_ROWS, OUT_CH), lambda i: (i, 0)),
            pl.BlockSpec((_ROWS, 16), lambda i: (i, 0)),
            pl.BlockSpec((_ROWS, 16), lambda i: (i, 0)),
        ],
        out_shape=[
            jax.ShapeDtypeStruct((PAD_N, OUT_CH), jnp.float32),
            jax.ShapeDtypeStruct((PAD_N, 16), jnp.float32),
            jax.ShapeDtypeStruct((PAD_N, 16), jnp.float32),
        ],
    )(p0, p1, b1r, W2, As2p, Ad2p)


def _final_body(p0_ref, p1_ref, b2_ref, o_ref):
    o_ref[...] = p0_ref[...] + p1_ref[...] + b2_ref[...]


def _final(p0, p1, b2r):
    return pl.pallas_call(
        _final_body,
        grid=(_GRID,),
        in_specs=[
            pl.BlockSpec((_ROWS, OUT_CH), lambda i: (i, 0)),
            pl.BlockSpec((_ROWS, OUT_CH), lambda i: (i, 0)),
            pl.BlockSpec((1, OUT_CH), lambda i: (0, 0)),
        ],
        out_specs=pl.BlockSpec((_ROWS, OUT_CH), lambda i: (i, 0)),
        out_shape=jax.ShapeDtypeStruct((PAD_N, OUT_CH), jnp.float32),
    )(p0, p1, b2r)


@jax.jit
def kernel(x, edge_index, W1, att_src1, att_dst1, bias1,
           W2, att_src2, att_dst2, bias2):
    ei = edge_index.astype(jnp.int32)
    loop = jnp.arange(N, dtype=jnp.int32)
    padlen = E_PAD - E_TOT
    pad = jnp.full((padlen,), N, jnp.int32)
    src = jnp.concatenate([ei[0], loop, pad])
    dst = jnp.concatenate([ei[1], loop, pad])

    D1 = HEADS * HID
    rows = jnp.arange(D1)
    cols = jnp.repeat(jnp.arange(HEADS), HID)
    As1p = jnp.zeros((D1, 16), jnp.float32).at[rows, cols].set(att_src1.reshape(-1))
    Ad1p = jnp.zeros((D1, 16), jnp.float32).at[rows, cols].set(att_dst1.reshape(-1))
    As2p = jnp.zeros((OUT_CH, 16), jnp.float32).at[:, 0].set(att_src2[0])
    Ad2p = jnp.zeros((OUT_CH, 16), jnp.float32).at[:, 0].set(att_dst2[0])

    x_pad = jnp.concatenate([x, jnp.zeros((PAD_N - N, IN_CH), jnp.float32)])
    z16 = jnp.zeros((PAD_N, 16), jnp.float32)
    z64 = jnp.zeros((PAD_N, OUT_CH), jnp.float32)

    h1, as1, ad1 = _proj1(x_pad, W1, As1p, Ad1p)
    expa1, denom1 = _sc_attn(src, dst, as1, ad1, z16)
    denr1 = _recip(denom1)
    (coef1,) = _sc_coef(dst, expa1, denr1)
    coef1d = coef1.reshape(-1)
    hks = [lax.slice(h1, (0, HID * k), (PAD_N, HID * (k + 1)))
           for k in range(HEADS)]
    (agg1,) = _sc_agg8(src, dst, coef1d, hks, z64)
    p1cat = agg1.transpose(1, 2, 0, 3).reshape(NC, PAD_N, HEADS * HID)

    h2, as2, ad2 = _proj2(p1cat[0], p1cat[1], bias1.reshape(1, -1),
                          W2, As2p, Ad2p)
    expa2, denom2 = _sc_attn(src, dst, as2, ad2, z16)
    denr2 = _recip(denom2)
    (coef2,) = _sc_coef(dst, expa2, denr2)
    (p2,) = _sc_agg(src, dst, coef2.reshape(-1), h2, z64, OUT_CH, 0)
    out = _final(p2[0], p2[1], bias2.reshape(1, -1))
    return out[:N]
